# baseline (speedup 1.0000x reference)
import functools

import jax
import jax.numpy as jnp
from jax import lax
from jax.experimental import pallas as pl
from jax.experimental.pallas import tpu as pltpu
from jax.experimental.pallas import tpu_sc as plsc

NC = 2
NS = 16
NW = NC * NS
L = 16

BATCH = 16384
D = 128
CB = 128
BPW = BATCH // NW
CH = BPW // CB
GROUPS = D // L


def _sc_partials(x, labels, centers):
  mesh = plsc.VectorSubcoreMesh(core_axis_name="c", subcore_axis_name="s")

  @functools.partial(
      pl.kernel,
      out_type=jax.ShapeDtypeStruct((NW, L), jnp.float32),
      mesh=mesh,
      scratch_types=[
          pltpu.VMEM((CH, CB), jnp.int32),
          pltpu.VMEM((2, CB, D), jnp.float32),
          pltpu.VMEM((2, CB, D), jnp.float32),
          pltpu.VMEM((L * L,), jnp.float32),
          pltpu.VMEM((L,), jnp.float32),
          pltpu.SemaphoreType.DMA,
          pltpu.SemaphoreType.DMA,
          pltpu.SemaphoreType.DMA,
          pltpu.SemaphoreType.DMA,
      ],
      compiler_params=pltpu.CompilerParams(needs_layout_passes=False),
  )
  def sc_kernel(x_hbm, lab_hbm, cen_hbm, out_hbm, idx_v, x_buf, c_buf,
                rs_buf, acc_v, semx0, semx1, semc0, semc1):
    wid = lax.axis_index("s") * NC + lax.axis_index("c")
    base = wid * BPW
    semx = [semx0, semx1]
    semc = [semc0, semc1]

    for kk in range(CH):
      pltpu.sync_copy(lab_hbm.at[pl.ds(base + kk * CB, CB)], idx_v.at[kk])

    def start(kk):
      sl = kk % 2
      hx = pltpu.async_copy(x_hbm.at[pl.ds(base + kk * CB, CB)],
                            x_buf.at[sl], semx[sl])
      hc = pltpu.async_copy(cen_hbm.at[idx_v.at[kk]], c_buf.at[sl], semc[sl])
      return hx, hc

    rows16 = jnp.arange(L, dtype=jnp.int32) * L

    def chunk_compute(acc, sl):
      def blk_body(b, acc):
        def samp(i, carry):
          s = b * L + i
          r = jnp.zeros((L,), jnp.float32)
          for g in range(GROUPS):
            dv = (x_buf[sl, s, pl.ds(g * L, L)]
                  - c_buf[sl, s, pl.ds(g * L, L)])
            r = r + dv * dv
          rs_buf[pl.ds(i * L, L)] = r
          return carry
        lax.fori_loop(0, L, samp, 0)
        dist = jnp.zeros((L,), jnp.float32)
        for col in range(L):
          dist = dist + plsc.load_gather(rs_buf, [rows16 + col])
        dist = jnp.minimum(jnp.maximum(dist, 1e-12), 1e12)
        return acc + dist
      return lax.fori_loop(0, CB // L, blk_body, acc)

    handles = start(0)
    acc = jnp.zeros((L,), jnp.float32)
    for kk in range(CH):
      hx, hc = handles
      if kk + 1 < CH:
        handles = start(kk + 1)
      hx.wait()
      hc.wait()
      acc = chunk_compute(acc, kk % 2)

    acc_v[...] = acc
    pltpu.sync_copy(acc_v, out_hbm.at[wid])

  return sc_kernel(x, labels, centers)


def _final_mean(partials):
  def body(p_ref, o_ref):
    o_ref[...] = jnp.sum(p_ref[...]).reshape(1, 1) * (1.0 / BATCH)

  return pl.pallas_call(
      body,
      out_shape=jax.ShapeDtypeStruct((1, 1), jnp.float32),
  )(partials)


def kernel(x, labels, centers):
  partials = _sc_partials(x, labels.astype(jnp.int32), centers)
  return _final_mean(partials)[0, 0]
